# baseline (speedup 1.0000x reference)
import functools

import jax
import jax.numpy as jnp
from jax import lax
from jax.experimental import pallas as pl
from jax.experimental.pallas import tpu as pltpu
from jax.experimental.pallas import tpu_sc as plsc

E = 8
TOPK = 2
BM = 256
RB = 512
SC_CH = 8
G_CH = 8
NW = 32


def _router_body(x_ref, gw_ref, logits_ref, w_ref, e_ref):
    x = x_ref[...]
    gw = gw_ref[...]
    logits = lax.dot_general(
        x.astype(jnp.bfloat16), gw.astype(jnp.bfloat16),
        (((1,), (1,)), ((), ())),
        preferred_element_type=jnp.float32,
    )
    logits_ref[...] = logits
    m = jnp.max(logits, axis=1, keepdims=True)
    ex = jnp.exp(logits - m)
    p = ex / jnp.sum(ex, axis=1, keepdims=True)
    a1 = jnp.argmax(p, axis=1).astype(jnp.int32)
    v1 = jnp.max(p, axis=1)
    cols = lax.broadcasted_iota(jnp.int32, p.shape, 1)
    p2 = jnp.where(cols == a1[:, None], -1.0, p)
    a2 = jnp.argmax(p2, axis=1).astype(jnp.int32)
    v2 = jnp.max(p2, axis=1)
    s = v1 + v2
    w_ref[...] = jnp.stack([v1 / s, v2 / s], axis=1)
    e_ref[...] = jnp.stack([a1, a2], axis=1)


def _router(x, gate_w):
    n = x.shape[0]
    grid = (n // RB,)
    return pl.pallas_call(
        _router_body,
        grid=grid,
        in_specs=[
            pl.BlockSpec((RB, x.shape[1]), lambda i: (i, 0)),
            pl.BlockSpec((E, x.shape[1]), lambda i: (0, 0)),
        ],
        out_specs=[
            pl.BlockSpec((RB, E), lambda i: (i, 0)),
            pl.BlockSpec((RB, TOPK), lambda i: (i, 0)),
            pl.BlockSpec((RB, TOPK), lambda i: (i, 0)),
        ],
        out_shape=[
            jax.ShapeDtypeStruct((n, E), jnp.float32),
            jax.ShapeDtypeStruct((n, TOPK), jnp.float32),
            jax.ShapeDtypeStruct((n, TOPK), jnp.int32),
        ],
    )(x, gate_w)


def _sc_gather_body(x_hbm, idx_hbm, out_hbm, idx_v, buf0, buf1, sem0, sem1):
    wid = lax.axis_index("s") * 2 + lax.axis_index("c")
    rows = idx_v.shape[0]
    nch = rows // G_CH
    base = wid * rows
    pltpu.sync_copy(idx_hbm.at[pl.ds(base, rows)], idx_v)
    bufs = (buf0, buf1)
    sems = (sem0, sem1)
    for b in range(2):
        pltpu.make_async_copy(
            x_hbm.at[idx_v.at[pl.ds(b * G_CH, G_CH)]], bufs[b], sems[b]
        ).start()

    @pl.loop(0, nch, step=2)
    def _(c):
        for b in range(2):
            cc = c + b
            pltpu.make_async_copy(
                x_hbm.at[idx_v.at[pl.ds(cc * G_CH, G_CH)]], bufs[b], sems[b]
            ).wait()
            pltpu.sync_copy(bufs[b], out_hbm.at[pl.ds(base + cc * G_CH, G_CH)])
            nxt = cc + 2

            @pl.when(nxt < nch)
            def _():
                pltpu.make_async_copy(
                    x_hbm.at[idx_v.at[pl.ds(nxt * G_CH, G_CH)]], bufs[b], sems[b]
                ).start()


def _sc_gather(x, idx):
    nt = idx.shape[0]
    d = x.shape[1]
    rows = nt // NW
    mesh = plsc.VectorSubcoreMesh(core_axis_name="c", subcore_axis_name="s")
    k = pl.kernel(
        _sc_gather_body,
        out_type=jax.ShapeDtypeStruct((nt, d), x.dtype),
        mesh=mesh,
        scratch_types=[
            pltpu.VMEM((rows,), jnp.int32),
            pltpu.VMEM((G_CH, d), x.dtype),
            pltpu.VMEM((G_CH, d), x.dtype),
            pltpu.SemaphoreType.DMA,
            pltpu.SemaphoreType.DMA,
        ],
    )
    return k(x, idx)


def _sc_combine_body(y_hbm, idx_hbm, out_hbm, idx_v, bufa, bufb, sema, semb):
    wid = lax.axis_index("s") * 2 + lax.axis_index("c")
    toks = (idx_v.shape[0] // (2 * SC_CH)) * SC_CH
    nch = toks // SC_CH
    d = bufa.shape[1]
    pltpu.sync_copy(idx_hbm.at[pl.ds(wid * 2 * toks, 2 * toks)], idx_v)
    pltpu.make_async_copy(
        y_hbm.at[idx_v.at[pl.ds(0, 2 * SC_CH)]], bufa, sema
    ).start()

    @pl.loop(0, nch, step=2)
    def _(c):
        pltpu.make_async_copy(
            y_hbm.at[idx_v.at[pl.ds((c + 1) * 2 * SC_CH, 2 * SC_CH)]], bufb, semb
        ).start()
        pltpu.make_async_copy(
            y_hbm.at[idx_v.at[pl.ds(c * 2 * SC_CH, 2 * SC_CH)]], bufa, sema
        ).wait()
        for r in range(SC_CH):
            @pl.loop(0, d, step=16)
            def _(i):
                bufa[r, pl.ds(i, 16)] = (
                    bufa[r, pl.ds(i, 16)] + bufa[r + SC_CH, pl.ds(i, 16)])
        pltpu.sync_copy(
            bufa.at[pl.ds(0, SC_CH)],
            out_hbm.at[pl.ds(wid * toks + c * SC_CH, SC_CH)])

        @pl.when(c + 2 < nch)
        def _():
            pltpu.make_async_copy(
                y_hbm.at[idx_v.at[pl.ds((c + 2) * 2 * SC_CH, 2 * SC_CH)]],
                bufa, sema
            ).start()

        pltpu.make_async_copy(
            y_hbm.at[idx_v.at[pl.ds((c + 1) * 2 * SC_CH, 2 * SC_CH)]], bufb, semb
        ).wait()
        for r in range(SC_CH):
            @pl.loop(0, d, step=16)
            def _(i):
                bufb[r, pl.ds(i, 16)] = (
                    bufb[r, pl.ds(i, 16)] + bufb[r + SC_CH, pl.ds(i, 16)])
        pltpu.sync_copy(
            bufb.at[pl.ds(0, SC_CH)],
            out_hbm.at[pl.ds(wid * toks + (c + 1) * SC_CH, SC_CH)])


def _sc_combine(y, idx_cat, t):
    d = y.shape[1]
    rows = t // NW
    mesh = plsc.VectorSubcoreMesh(core_axis_name="c", subcore_axis_name="s")
    k = pl.kernel(
        _sc_combine_body,
        out_type=jax.ShapeDtypeStruct((t, d), jnp.float32),
        mesh=mesh,
        scratch_types=[
            pltpu.VMEM((2 * rows,), jnp.int32),
            pltpu.VMEM((2 * SC_CH, d), jnp.float32),
            pltpu.VMEM((2 * SC_CH, d), jnp.float32),
            pltpu.SemaphoreType.DMA,
            pltpu.SemaphoreType.DMA,
        ],
    )
    return k(y, idx_cat)


def _mlp_gu_body(meta_ref, xs_ref, wg_ref, wu_ref, h_ref, *, b0=0):
    j = pl.program_id(0)
    b = meta_ref[0, j]
    lo = meta_ref[2, j]
    hi = meta_ref[3, j]

    @pl.when(hi > lo)
    def _():
        pos = b * BM + lax.broadcasted_iota(jnp.int32, (BM, 1), 0)
        mask = (pos >= lo) & (pos < hi)
        xm = jnp.where(mask, xs_ref[...], 0.0).astype(jnp.bfloat16)
        g = lax.dot_general(
            xm, wg_ref[0], (((1,), (1,)), ((), ())),
            preferred_element_type=jnp.float32,
        )
        u = lax.dot_general(
            xm, wu_ref[0], (((1,), (1,)), ((), ())),
            preferred_element_type=jnp.float32,
        )
        h = ((g * jax.nn.sigmoid(g)) * u).astype(jnp.bfloat16)
        first = lo == b * BM

        @pl.when(first)
        def _():
            h_ref[...] = h

        @pl.when(jnp.logical_not(first))
        def _():
            h_ref[...] = h_ref[...] + h


def _mlp_down_body(meta_ref, h_ref, wd_ref, ws_ref, y_ref):
    j = pl.program_id(0)
    b = meta_ref[0, j]
    lo = meta_ref[2, j]
    hi = meta_ref[3, j]

    @pl.when(hi > lo)
    def _():
        pos = b * BM + lax.broadcasted_iota(jnp.int32, (BM, 1), 0)
        mask = (pos >= lo) & (pos < hi)
        h = jnp.where(mask, h_ref[...], 0)
        y = lax.dot_general(
            h, wd_ref[0], (((1,), (1,)), ((), ())),
            preferred_element_type=jnp.float32,
        )
        y = y * ws_ref[0, 0].reshape(BM, 1)
        first = lo == b * BM

        @pl.when(first)
        def _():
            y_ref[...] = y

        @pl.when(jnp.logical_not(first))
        def _():
            y_ref[...] = y_ref[...] + y


def _mlp_gu_body_alias(meta_ref, xs_ref, wg_ref, wu_ref, hprev_ref, h_ref,
                       *, b0=0):
    del hprev_ref
    _mlp_gu_body(meta_ref, xs_ref, wg_ref, wu_ref, h_ref, b0=b0)


def _mlp(xs0, xs1, wg, wu, wd, ws, meta0, meta1, meta, nt, ntiles_h, ntiles):
    d = xs0.shape[1]
    dff = wg.shape[1]
    nblk = nt // BM
    hblk = nblk // 2

    def gu_specs(b0):
        return [
            pl.BlockSpec((BM, d), lambda j, m: (m[0, j] - b0, 0)),
            pl.BlockSpec((1, dff, d), lambda j, m: (m[1, j], 0, 0)),
            pl.BlockSpec((1, dff, d), lambda j, m: (m[1, j], 0, 0)),
        ]

    h = pl.pallas_call(
        _mlp_gu_body,
        grid_spec=pltpu.PrefetchScalarGridSpec(
            num_scalar_prefetch=1,
            grid=(ntiles_h,),
            in_specs=gu_specs(0),
            out_specs=pl.BlockSpec((BM, dff), lambda j, m: (m[0, j], 0)),
        ),
        out_shape=jax.ShapeDtypeStruct((nt, dff), jnp.bfloat16),
    )(meta0, xs0, wg, wu)
    h = pl.pallas_call(
        functools.partial(_mlp_gu_body_alias, b0=hblk),
        grid_spec=pltpu.PrefetchScalarGridSpec(
            num_scalar_prefetch=1,
            grid=(ntiles_h,),
            in_specs=gu_specs(hblk)
            + [pl.BlockSpec(memory_space=pltpu.MemorySpace.HBM)],
            out_specs=pl.BlockSpec((BM, dff), lambda j, m: (m[0, j], 0)),
        ),
        out_shape=jax.ShapeDtypeStruct((nt, dff), jnp.bfloat16),
        input_output_aliases={4: 0},
    )(meta1, xs1, wg, wu, h)
    down_spec = pltpu.PrefetchScalarGridSpec(
        num_scalar_prefetch=1,
        grid=(ntiles,),
        in_specs=[
            pl.BlockSpec((BM, dff), lambda j, m: (m[0, j], 0)),
            pl.BlockSpec((1, d, dff), lambda j, m: (m[1, j], 0, 0)),
            pl.BlockSpec((1, 1, BM), lambda j, m: (m[0, j], 0, 0)),
        ],
        out_specs=pl.BlockSpec((BM, d), lambda j, m: (m[0, j], 0)),
    )
    return pl.pallas_call(
        _mlp_down_body,
        grid_spec=down_spec,
        out_shape=jax.ShapeDtypeStruct((nt, d), jnp.float32),
    )(meta, h, wd, ws.reshape(nblk, 1, BM))


def _tile_meta(counts, b0, b1, ntiles):
    nb = b1 - b0
    offsets = jnp.concatenate([jnp.zeros((1,), jnp.int32), jnp.cumsum(counts)])
    starts = offsets[:-1]
    ends = offsets[1:]
    barange = b0 + jnp.arange(nb, dtype=jnp.int32)
    overlap = ((starts[None, :] < (barange[:, None] + 1) * BM)
               & (ends[None, :] > barange[:, None] * BM))
    flat = overlap.reshape(-1)
    (active,) = jnp.nonzero(flat, size=ntiles, fill_value=0)
    n_act = jnp.sum(flat.astype(jnp.int32))
    jar = jnp.arange(ntiles, dtype=jnp.int32)
    is_real = jar < n_act
    last_flat = jnp.max(jnp.where(flat, jnp.arange(nb * E, dtype=jnp.int32), -1))
    afi = jnp.where(is_real, active.astype(jnp.int32), last_flat)
    tb = b0 + afi // E
    te = afi % E
    tlo = jnp.where(is_real, jnp.maximum(starts[te], tb * BM), 0)
    thi = jnp.where(is_real, jnp.minimum(ends[te], (tb + 1) * BM), 0)
    return jnp.stack([tb, te, tlo, thi], axis=0)


def kernel(hidden_states, gate_w, gate_proj_w, up_proj_w, down_proj_w):
    b, s, d = hidden_states.shape
    n = b * s
    nt = n * TOPK
    nblk = nt // BM
    ntiles = nblk + E - 1
    x = hidden_states.reshape(n, d)

    logits, w_pair, e_pair = _router(x, gate_w)

    e_flat = e_pair.reshape(-1)

    counts = jnp.sum(e_flat[:, None] == jnp.arange(E, dtype=jnp.int32)[None, :],
                     axis=0, dtype=jnp.int32)
    ar = jnp.arange(nt, dtype=jnp.int32)
    perm = jnp.sort(e_flat * 16384 + ar) & 16383
    tok_sorted = (perm // TOPK).astype(jnp.int32)
    w_sorted = w_pair.reshape(-1)[perm]
    pos = jnp.zeros((nt,), jnp.int32).at[perm].set(ar)
    ipos = pos.reshape(n, TOPK)
    idx_cat = jnp.concatenate(
        [ipos[:, 0].reshape(-1, SC_CH), ipos[:, 1].reshape(-1, SC_CH)],
        axis=1).reshape(-1)
    hblk = nblk // 2
    ntiles_h = hblk + E - 1
    meta0 = _tile_meta(counts, 0, hblk, ntiles_h)
    meta1 = _tile_meta(counts, hblk, nblk, ntiles_h)
    meta = _tile_meta(counts, 0, nblk, ntiles)

    half = nt // 2
    xs0 = _sc_gather(x, tok_sorted[:half])
    xs1 = _sc_gather(x, tok_sorted[half:])
    ys = _mlp(xs0, xs1, gate_proj_w, up_proj_w, down_proj_w, w_sorted,
              meta0, meta1, meta, nt, ntiles_h, ntiles)
    final = _sc_combine(ys, idx_cat, n)

    return final.reshape(b, s, d), logits


# speedup vs baseline: 1.0568x; 1.0568x over previous
import functools

import jax
import jax.numpy as jnp
from jax import lax
from jax.experimental import pallas as pl
from jax.experimental.pallas import tpu as pltpu
from jax.experimental.pallas import tpu_sc as plsc

E = 8
TOPK = 2
BM = 256
RB = 512
SC_CH = 8
G_CH = 8
NW = 32


def _router_body(x_ref, gw_ref, logits_ref, w_ref, e_ref):
    x = x_ref[...]
    gw = gw_ref[...]
    logits = lax.dot_general(
        x.astype(jnp.bfloat16), gw.astype(jnp.bfloat16),
        (((1,), (1,)), ((), ())),
        preferred_element_type=jnp.float32,
    )
    logits_ref[...] = logits
    m = jnp.max(logits, axis=1, keepdims=True)
    ex = jnp.exp(logits - m)
    p = ex / jnp.sum(ex, axis=1, keepdims=True)
    a1 = jnp.argmax(p, axis=1).astype(jnp.int32)
    v1 = jnp.max(p, axis=1)
    cols = lax.broadcasted_iota(jnp.int32, p.shape, 1)
    p2 = jnp.where(cols == a1[:, None], -1.0, p)
    a2 = jnp.argmax(p2, axis=1).astype(jnp.int32)
    v2 = jnp.max(p2, axis=1)
    s = v1 + v2
    w_ref[...] = jnp.stack([v1 / s, v2 / s], axis=1)
    e_ref[...] = jnp.stack([a1, a2], axis=1)


def _router(x, gate_w):
    n = x.shape[0]
    grid = (n // RB,)
    return pl.pallas_call(
        _router_body,
        grid=grid,
        in_specs=[
            pl.BlockSpec((RB, x.shape[1]), lambda i: (i, 0)),
            pl.BlockSpec((E, x.shape[1]), lambda i: (0, 0)),
        ],
        out_specs=[
            pl.BlockSpec((RB, E), lambda i: (i, 0)),
            pl.BlockSpec((RB, TOPK), lambda i: (i, 0)),
            pl.BlockSpec((RB, TOPK), lambda i: (i, 0)),
        ],
        out_shape=[
            jax.ShapeDtypeStruct((n, E), jnp.float32),
            jax.ShapeDtypeStruct((n, TOPK), jnp.float32),
            jax.ShapeDtypeStruct((n, TOPK), jnp.int32),
        ],
    )(x, gate_w)


def _sc_gather_body(x_hbm, idx_hbm, out_hbm, idx_v, buf0, buf1, sem0, sem1):
    wid = lax.axis_index("s") * 2 + lax.axis_index("c")
    rows = idx_v.shape[0]
    nch = rows // G_CH
    base = wid * rows
    pltpu.sync_copy(idx_hbm.at[pl.ds(base, rows)], idx_v)
    bufs = (buf0, buf1)
    sems = (sem0, sem1)
    for b in range(2):
        pltpu.make_async_copy(
            x_hbm.at[idx_v.at[pl.ds(b * G_CH, G_CH)]], bufs[b], sems[b]
        ).start()

    @pl.loop(0, nch, step=2)
    def _(c):
        for b in range(2):
            cc = c + b
            pltpu.make_async_copy(
                x_hbm.at[idx_v.at[pl.ds(cc * G_CH, G_CH)]], bufs[b], sems[b]
            ).wait()
            pltpu.sync_copy(bufs[b], out_hbm.at[pl.ds(base + cc * G_CH, G_CH)])
            nxt = cc + 2

            @pl.when(nxt < nch)
            def _():
                pltpu.make_async_copy(
                    x_hbm.at[idx_v.at[pl.ds(nxt * G_CH, G_CH)]], bufs[b], sems[b]
                ).start()


def _sc_gather(x, idx):
    nt = idx.shape[0]
    d = x.shape[1]
    rows = nt // NW
    mesh = plsc.VectorSubcoreMesh(core_axis_name="c", subcore_axis_name="s")
    k = pl.kernel(
        _sc_gather_body,
        out_type=jax.ShapeDtypeStruct((nt, d), x.dtype),
        mesh=mesh,
        scratch_types=[
            pltpu.VMEM((rows,), jnp.int32),
            pltpu.VMEM((G_CH, d), x.dtype),
            pltpu.VMEM((G_CH, d), x.dtype),
            pltpu.SemaphoreType.DMA,
            pltpu.SemaphoreType.DMA,
        ],
    )
    return k(x, idx)


def _sc_combine_body(y_hbm, idx_hbm, out_hbm, idx_v, bufa, bufb, sema, semb):
    wid = lax.axis_index("s") * 2 + lax.axis_index("c")
    toks = (idx_v.shape[0] // (2 * SC_CH)) * SC_CH
    nch = toks // SC_CH
    d = bufa.shape[1]
    pltpu.sync_copy(idx_hbm.at[pl.ds(wid * 2 * toks, 2 * toks)], idx_v)
    pltpu.make_async_copy(
        y_hbm.at[idx_v.at[pl.ds(0, 2 * SC_CH)]], bufa, sema
    ).start()

    @pl.loop(0, nch, step=2)
    def _(c):
        pltpu.make_async_copy(
            y_hbm.at[idx_v.at[pl.ds((c + 1) * 2 * SC_CH, 2 * SC_CH)]], bufb, semb
        ).start()
        pltpu.make_async_copy(
            y_hbm.at[idx_v.at[pl.ds(c * 2 * SC_CH, 2 * SC_CH)]], bufa, sema
        ).wait()
        for r in range(SC_CH):
            @pl.loop(0, d, step=16)
            def _(i):
                bufa[r, pl.ds(i, 16)] = (
                    bufa[r, pl.ds(i, 16)] + bufa[r + SC_CH, pl.ds(i, 16)])
        pltpu.sync_copy(
            bufa.at[pl.ds(0, SC_CH)],
            out_hbm.at[pl.ds(wid * toks + c * SC_CH, SC_CH)])

        @pl.when(c + 2 < nch)
        def _():
            pltpu.make_async_copy(
                y_hbm.at[idx_v.at[pl.ds((c + 2) * 2 * SC_CH, 2 * SC_CH)]],
                bufa, sema
            ).start()

        pltpu.make_async_copy(
            y_hbm.at[idx_v.at[pl.ds((c + 1) * 2 * SC_CH, 2 * SC_CH)]], bufb, semb
        ).wait()
        for r in range(SC_CH):
            @pl.loop(0, d, step=16)
            def _(i):
                bufb[r, pl.ds(i, 16)] = (
                    bufb[r, pl.ds(i, 16)] + bufb[r + SC_CH, pl.ds(i, 16)])
        pltpu.sync_copy(
            bufb.at[pl.ds(0, SC_CH)],
            out_hbm.at[pl.ds(wid * toks + (c + 1) * SC_CH, SC_CH)])


def _sc_combine(y, idx_cat, t):
    d = y.shape[1]
    rows = t // NW
    mesh = plsc.VectorSubcoreMesh(core_axis_name="c", subcore_axis_name="s")
    k = pl.kernel(
        _sc_combine_body,
        out_type=jax.ShapeDtypeStruct((t, d), jnp.float32),
        mesh=mesh,
        scratch_types=[
            pltpu.VMEM((2 * rows,), jnp.int32),
            pltpu.VMEM((2 * SC_CH, d), jnp.float32),
            pltpu.VMEM((2 * SC_CH, d), jnp.float32),
            pltpu.SemaphoreType.DMA,
            pltpu.SemaphoreType.DMA,
        ],
    )
    return k(y, idx_cat)


def _mlp_gu_body(meta_ref, xs_ref, wg_ref, wu_ref, h_ref):
    j = pl.program_id(0)
    b = meta_ref[0, j]
    lo = meta_ref[2, j]
    hi = meta_ref[3, j]

    @pl.when(hi > lo)
    def _():
        pos = b * BM + lax.broadcasted_iota(jnp.int32, (BM, 1), 0)
        mask = (pos >= lo) & (pos < hi)
        xm = jnp.where(mask, xs_ref[...], 0.0).astype(jnp.bfloat16)
        g = lax.dot_general(
            xm, wg_ref[0], (((1,), (1,)), ((), ())),
            preferred_element_type=jnp.float32,
        )
        u = lax.dot_general(
            xm, wu_ref[0], (((1,), (1,)), ((), ())),
            preferred_element_type=jnp.float32,
        )
        h = ((g * jax.nn.sigmoid(g)) * u).astype(jnp.bfloat16)
        first = lo == b * BM

        @pl.when(first)
        def _():
            h_ref[...] = h

        @pl.when(jnp.logical_not(first))
        def _():
            h_ref[...] = h_ref[...] + h


def _mlp_down_body(meta_ref, h_ref, wd_ref, ws_ref, y_ref):
    j = pl.program_id(0)
    b = meta_ref[0, j]
    lo = meta_ref[2, j]
    hi = meta_ref[3, j]

    @pl.when(hi > lo)
    def _():
        pos = b * BM + lax.broadcasted_iota(jnp.int32, (BM, 1), 0)
        mask = (pos >= lo) & (pos < hi)
        h = jnp.where(mask, h_ref[...], 0)
        y = lax.dot_general(
            h, wd_ref[0], (((1,), (1,)), ((), ())),
            preferred_element_type=jnp.float32,
        )
        y = y * ws_ref[0, 0].reshape(BM, 1)
        first = lo == b * BM

        @pl.when(first)
        def _():
            y_ref[...] = y

        @pl.when(jnp.logical_not(first))
        def _():
            y_ref[...] = y_ref[...] + y


def _mlp(xs, wg, wu, wd, ws, meta, ntiles):
    nt, d = xs.shape
    dff = wg.shape[1]
    nblk = nt // BM
    gu_spec = pltpu.PrefetchScalarGridSpec(
        num_scalar_prefetch=1,
        grid=(ntiles,),
        in_specs=[
            pl.BlockSpec((BM, d), lambda j, m: (m[0, j], 0)),
            pl.BlockSpec((1, dff, d), lambda j, m: (m[1, j], 0, 0)),
            pl.BlockSpec((1, dff, d), lambda j, m: (m[1, j], 0, 0)),
        ],
        out_specs=pl.BlockSpec((BM, dff), lambda j, m: (m[0, j], 0)),
    )
    h = pl.pallas_call(
        _mlp_gu_body,
        grid_spec=gu_spec,
        out_shape=jax.ShapeDtypeStruct((nt, dff), jnp.bfloat16),
    )(meta, xs, wg, wu)
    down_spec = pltpu.PrefetchScalarGridSpec(
        num_scalar_prefetch=1,
        grid=(ntiles,),
        in_specs=[
            pl.BlockSpec((BM, dff), lambda j, m: (m[0, j], 0)),
            pl.BlockSpec((1, d, dff), lambda j, m: (m[1, j], 0, 0)),
            pl.BlockSpec((1, 1, BM), lambda j, m: (m[0, j], 0, 0)),
        ],
        out_specs=pl.BlockSpec((BM, d), lambda j, m: (m[0, j], 0)),
    )
    return pl.pallas_call(
        _mlp_down_body,
        grid_spec=down_spec,
        out_shape=jax.ShapeDtypeStruct((nt, d), jnp.float32),
    )(meta, h, wd, ws.reshape(nblk, 1, BM))


def _tile_meta(counts, nblk, ntiles):
    offsets = jnp.concatenate([jnp.zeros((1,), jnp.int32), jnp.cumsum(counts)])
    starts = offsets[:-1]
    ends = offsets[1:]
    barange = jnp.arange(nblk, dtype=jnp.int32)
    overlap = ((starts[None, :] < (barange[:, None] + 1) * BM)
               & (ends[None, :] > barange[:, None] * BM))
    flat = overlap.reshape(-1)
    (active,) = jnp.nonzero(flat, size=ntiles, fill_value=0)
    n_act = jnp.sum(flat.astype(jnp.int32))
    jar = jnp.arange(ntiles, dtype=jnp.int32)
    is_real = jar < n_act
    last_flat = jnp.max(jnp.where(flat, jnp.arange(nblk * E, dtype=jnp.int32), -1))
    afi = jnp.where(is_real, active.astype(jnp.int32), last_flat)
    tb = afi // E
    te = afi % E
    tlo = jnp.where(is_real, jnp.maximum(starts[te], tb * BM), 0)
    thi = jnp.where(is_real, jnp.minimum(ends[te], (tb + 1) * BM), 0)
    return jnp.stack([tb, te, tlo, thi], axis=0)


def kernel(hidden_states, gate_w, gate_proj_w, up_proj_w, down_proj_w):
    b, s, d = hidden_states.shape
    n = b * s
    nt = n * TOPK
    nblk = nt // BM
    ntiles = nblk + E - 1
    x = hidden_states.reshape(n, d)

    logits, w_pair, e_pair = _router(x, gate_w)

    e_flat = e_pair.reshape(-1)

    counts = jnp.sum(e_flat[:, None] == jnp.arange(E, dtype=jnp.int32)[None, :],
                     axis=0, dtype=jnp.int32)
    ar = jnp.arange(nt, dtype=jnp.int32)
    perm = jnp.sort(e_flat * 16384 + ar) & 16383
    tok_sorted = (perm // TOPK).astype(jnp.int32)
    w_sorted = w_pair.reshape(-1)[perm]
    pos = jnp.zeros((nt,), jnp.int32).at[perm].set(ar)
    ipos = pos.reshape(n, TOPK)
    idx_cat = jnp.concatenate(
        [ipos[:, 0].reshape(-1, SC_CH), ipos[:, 1].reshape(-1, SC_CH)],
        axis=1).reshape(-1)
    meta = _tile_meta(counts, nblk, ntiles)

    xs = _sc_gather(x, tok_sorted)
    ys = _mlp(xs, gate_proj_w, up_proj_w, down_proj_w, w_sorted, meta, ntiles)
    final = _sc_combine(ys, idx_cat, n)

    return final.reshape(b, s, d), logits
